# Initial kernel scaffold; baseline (speedup 1.0000x reference)
import jax
import jax.numpy as jnp
from jax.experimental import pallas as pl


def kernel(x_Tm1, log_w, y_T, noise, u, A, C, proc_log_scale, obs_log_scale):
    raise NotImplementedError("write your pallas kernel here")


# speedup vs baseline: 1.5518x; 1.5518x over previous
import jax
import jax.numpy as jnp
import numpy as np
from jax.experimental import pallas as pl
from jax.experimental.pallas import tpu as pltpu

_N = 1000000
_CH = 8192
_STEPS = 123
_NP = _CH * _STEPS
_LOG2PI = float(np.log(2.0 * np.pi))


def _pf_kernel(yT_ref, A_ref, C_ref, sc_ref, xres_ref, noise_ref, bias_ref, out_ref,
               m_ref, S_ref, V_ref, M2_ref):
    i = pl.program_id(0)

    @pl.when(i == 0)
    def _init():
        m_ref[0, 0] = -1e30
        S_ref[0, 0] = 0.0
        V_ref[...] = jnp.zeros_like(V_ref)
        M2_ref[...] = jnp.zeros_like(M2_ref)

    proc_ls = sc_ref[0, 0]
    obs_ls = sc_ref[0, 1]

    x = xres_ref[...]
    nz = noise_ref[...]
    xT = jnp.dot(A_ref[...], x, preferred_element_type=jnp.float32)
    xT = xT + nz * jnp.exp(proc_ls)

    ym = jnp.dot(C_ref[...], xT, preferred_element_type=jnp.float32)
    s = jnp.exp(obs_ls)
    d = (yT_ref[...] - ym) / s
    lp = jnp.sum(-0.5 * d * d, axis=0, keepdims=True) \
        - 4.0 * obs_ls - 2.0 * _LOG2PI + bias_ref[...]

    m_old = m_ref[0, 0]
    m_new = jnp.maximum(m_old, jnp.max(lp))
    scale = jnp.exp(m_old - m_new)
    p = jnp.exp(lp - m_new)

    S_ref[0, 0] = S_ref[0, 0] * scale + jnp.sum(p)
    V_ref[...] = V_ref[...] * scale + jnp.sum(xT * p, axis=1, keepdims=True)
    xTp = xT * p
    M2_ref[...] = M2_ref[...] * scale + jnp.dot(
        xTp, xT.T, preferred_element_type=jnp.float32)
    m_ref[0, 0] = m_new

    @pl.when(i == _STEPS - 1)
    def _finish():
        Sv = S_ref[0, 0]
        mu = V_ref[...] / Sv
        cov = M2_ref[...] / Sv - jnp.dot(mu, mu.T,
                                         preferred_element_type=jnp.float32)
        out_ref[...] = jnp.concatenate([mu, cov], axis=1)


def kernel(x_Tm1, log_w, y_T, noise, u, A, C, proc_log_scale, obs_log_scale):
    w = jax.nn.softmax(log_w[:, 0])
    cumw = jnp.cumsum(w)
    anc = jnp.clip(jnp.searchsorted(cumw, u), 0, _N - 1)
    pad = _NP - _N
    x_res = jnp.take(x_Tm1, anc, axis=0).T
    x_res = jnp.pad(x_res, ((0, 0), (0, pad)))
    noise_t = jnp.pad(noise.T, ((0, 0), (0, pad)))
    bias = jnp.concatenate([jnp.zeros((1, _N), jnp.float32),
                            jnp.full((1, pad), -1e30, jnp.float32)], axis=1)
    sc = jnp.stack([proc_log_scale, obs_log_scale]).reshape(1, 2)

    out89 = pl.pallas_call(
        _pf_kernel,
        grid=(_STEPS,),
        in_specs=[
            pl.BlockSpec((4, 1), lambda i: (0, 0)),
            pl.BlockSpec((8, 8), lambda i: (0, 0)),
            pl.BlockSpec((4, 8), lambda i: (0, 0)),
            pl.BlockSpec((1, 2), lambda i: (0, 0)),
            pl.BlockSpec((8, _CH), lambda i: (0, i)),
            pl.BlockSpec((8, _CH), lambda i: (0, i)),
            pl.BlockSpec((1, _CH), lambda i: (0, i)),
        ],
        out_specs=pl.BlockSpec((8, 9), lambda i: (0, 0)),
        out_shape=jax.ShapeDtypeStruct((8, 9), jnp.float32),
        scratch_shapes=[
            pltpu.SMEM((1, 1), jnp.float32),
            pltpu.SMEM((1, 1), jnp.float32),
            pltpu.VMEM((8, 1), jnp.float32),
            pltpu.VMEM((8, 8), jnp.float32),
        ],
    )(y_T.reshape(4, 1), A, C, sc, x_res, noise_t, bias)

    mu = out89[:, 0]
    cov = out89[:, 1:]
    return jnp.concatenate([mu, cov.reshape(-1)])
